# baseline (speedup 1.0000x reference)
import functools

import jax
import jax.numpy as jnp
from jax import lax
from jax.experimental import pallas as pl
from jax.experimental.pallas import tpu as pltpu
from jax.experimental.pallas import tpu_sc as plsc

N = 10000
E = 320000
D_IN = 128
H1 = 128
H2 = 64

NC = 2
NS = 16
NW = NC * NS
CHUNK = 128
CPS = 160
E_PAD = NS * CPS * CHUNK
NPAD = N + 8
NBUF = 4
ABUF = 4
_DIAG = 2
STRIPE_A = 632
STRIPE_B = N - 15 * STRIPE_A

_mesh = plsc.VectorSubcoreMesh(core_axis_name="c", subcore_axis_name="s")
_sc_params = pltpu.CompilerParams(use_tc_tiling_on_sc=False)


def _stripe_sizes(tile15):
    total = STRIPE_B if tile15 else STRIPE_A
    sizes = []
    while total:
        k = min(CHUNK, total)
        sizes.append(k)
        total -= k
    return sizes


def _stripe_pieces(piece_fn, s):
    @pl.when(s < 15)
    def _():
        off = 0
        for k in _stripe_sizes(False):
            piece_fn(s * STRIPE_A + off, k)
            off += k
    @pl.when(s == 15)
    def _():
        off = 0
        for k in _stripe_sizes(True):
            piece_fn(15 * STRIPE_A + off, k)
            off += k


def _make_deg_kernel():
    @functools.partial(
        pl.kernel,
        out_type=(jax.ShapeDtypeStruct((N,), jnp.float32),
                  jax.ShapeDtypeStruct((N,), jnp.float32)),
        mesh=_mesh,
        compiler_params=_sc_params,
        scratch_types=[
            pltpu.VMEM((CPS // NC, CHUNK), jnp.int32),
            pltpu.VMEM((CHUNK,), jnp.float32),
            pltpu.VMEM((640,), jnp.float32),
            pltpu.VMEM_SHARED((NPAD,), jnp.float32),
            pltpu.SemaphoreType.DMA,
            pltpu.SemaphoreType.DMA,
            pltpu.SemaphoreType.DMA,
            pltpu.SemaphoreType.DMA,
            pltpu.SemaphoreType.DMA,
        ],
    )
    def deg_kernel(dst3d, out0, out1, didx, ones, zbuf, acc,
                   isem, s0, s1, s2, s3):
        ssem = [s0, s1, s2, s3]
        c = lax.axis_index("c")
        s = lax.axis_index("s")
        wid = s * NC + c

        idesc = pltpu.async_copy(dst3d.at[wid], didx, isem)
        for g in range(8):
            ones[pl.ds(g * 16, 16)] = jnp.ones((16,), jnp.float32)
        for g in range(40):
            zbuf[pl.ds(g * 16, 16)] = jnp.zeros((16,), jnp.float32)
        _stripe_pieces(lambda o, n: pltpu.sync_copy(
            zbuf.at[pl.ds(0, n)], acc.at[pl.ds(o, n)]), s)
        idesc.wait()
        plsc.subcore_barrier()

        def group(g, carry):
            descs = []
            for b in range(NBUF):
                descs.append(pltpu.async_copy(
                    ones, acc.at[didx.at[g * NBUF + b]], ssem[b], add=True))
            for b in range(NBUF):
                descs[b].wait()
            return carry
        lax.fori_loop(0, CPS // NC // NBUF, group, 0)
        plsc.subcore_barrier()

        def bounce(outref):
            def piece(o, n):
                pltpu.sync_copy(acc.at[pl.ds(o, n)], zbuf.at[pl.ds(0, n)])
                pltpu.sync_copy(zbuf.at[pl.ds(0, n)], outref.at[pl.ds(o, n)])
            return piece
        @pl.when(c == 0)
        def _():
            _stripe_pieces(bounce(out0), s)
        @pl.when(c == 1)
        def _():
            _stripe_pieces(bounce(out1), s)

    return deg_kernel


def _make_agg_kernel(DH):
    @functools.partial(
        pl.kernel,
        out_type=jax.ShapeDtypeStruct((NC, N, DH), jnp.float32),
        mesh=_mesh,
        compiler_params=_sc_params,
        scratch_types=[
            pltpu.VMEM((CPS, CHUNK), jnp.int32),
            pltpu.VMEM((CPS, CHUNK), jnp.int32),
            pltpu.VMEM((ABUF, CHUNK, DH), jnp.float32),
            pltpu.VMEM_SHARED((NPAD, DH), jnp.float32),
            pltpu.SemaphoreType.DMA,
        ] + [pltpu.SemaphoreType.DMA] * (2 * ABUF),
    )
    def agg_kernel(h_split, src3d, dst3d, out,
                   sidx, didx, buf, acc, isem, *sems):
        gsem = list(sems[:ABUF])
        ssem = list(sems[ABUF:])
        c = lax.axis_index("c")
        s = lax.axis_index("s")
        h_c = h_split.at[c]

        d1 = pltpu.async_copy(src3d.at[s], sidx, isem)
        d2 = pltpu.async_copy(dst3d.at[s], didx, isem)
        def zrow(i, carry):
            for g in range(DH // 16):
                buf[0, i, pl.ds(g * 16, 16)] = jnp.zeros((16,), jnp.float32)
            return carry
        lax.fori_loop(0, CHUNK, zrow, 0)
        _stripe_pieces(lambda o, n: pltpu.sync_copy(
            buf.at[0, pl.ds(0, n)], acc.at[pl.ds(o, n)]), s)
        d1.wait()
        d2.wait()
        plsc.subcore_barrier()

        def fire_scatter(b, j):
            if _DIAG == 1:
                pltpu.async_copy(
                    buf.at[b], acc.at[pl.ds(0, CHUNK)], ssem[b])
            else:
                pltpu.async_copy(
                    buf.at[b], acc.at[didx.at[j]], ssem[b], add=True)

        def drain_scatter(b, j):
            if _DIAG == 1:
                pltpu.make_async_copy(
                    buf.at[b], acc.at[pl.ds(0, CHUNK)], ssem[b]).wait()
            else:
                pltpu.make_async_copy(
                    buf.at[b], acc.at[didx.at[j]], ssem[b]).wait()

        def fire_gather(b, j):
            if _DIAG == 2:
                return pltpu.async_copy(
                    h_c.at[pl.ds(0, CHUNK)], buf.at[b], gsem[b])
            return pltpu.async_copy(h_c.at[sidx.at[j]], buf.at[b], gsem[b])

        def group(g, carry):
            gd = []
            for b in range(ABUF):
                @pl.when(g > 0)
                def _(b=b):
                    drain_scatter(b, (g - 1) * ABUF + b)
                gd.append(fire_gather(b, g * ABUF + b))
            for b in range(ABUF):
                gd[b].wait()
                fire_scatter(b, g * ABUF + b)
            return carry
        ng = CPS // ABUF
        lax.fori_loop(0, ng, group, 0)
        for b in range(ABUF):
            drain_scatter(b, (ng - 1) * ABUF + b)
        plsc.subcore_barrier()

        def piece(o, n):
            pltpu.sync_copy(acc.at[pl.ds(o, n)], buf.at[0, pl.ds(0, n)])
            pltpu.sync_copy(buf.at[0, pl.ds(0, n)], out.at[c, pl.ds(o, n)])
        _stripe_pieces(piece, s)

    return agg_kernel


_deg_kernel = _make_deg_kernel()
_agg64 = _make_agg_kernel(H1 // NC)
_agg32 = _make_agg_kernel(H2 // NC)

BR = 1000
GRID = N // BR


def _dinv_block(degp_ref):
    p = degp_ref[...]
    return lax.rsqrt(1.0 + p[0, 0] + p[0, 1])


def _mm1_body(x_ref, w_ref, o_ref):
    o_ref[...] = jnp.dot(x_ref[...], w_ref[...],
                         preferred_element_type=jnp.float32)


def _scale_split_body(h_ref, degp_ref, o_ref):
    dinv = _dinv_block(degp_ref)
    hp = h_ref[...] * dinv[:, None]
    d = hp.shape[1] // 2
    o_ref[0] = hp[:, :d]
    o_ref[1] = hp[:, d:]


def _layer2_body(q_ref, hp_ref, degp_ref, w_ref, o_ref):
    dinv = _dinv_block(degp_ref)
    agg = jnp.concatenate([q_ref[0] + hp_ref[0], q_ref[1] + hp_ref[1]],
                          axis=1)
    z1 = jnp.maximum(agg * dinv[:, None], 0.0)
    h2 = jnp.dot(z1, w_ref[...], preferred_element_type=jnp.float32)
    hp2 = h2 * dinv[:, None]
    d = hp2.shape[1] // 2
    o_ref[0] = hp2[:, :d]
    o_ref[1] = hp2[:, d:]


def _final_body(r_ref, hp_ref, degp_ref, o_ref):
    dinv = _dinv_block(degp_ref)
    agg = jnp.concatenate([r_ref[0] + hp_ref[0], r_ref[1] + hp_ref[1]],
                          axis=1)
    o_ref[...] = agg * dinv[:, None]


def _rows_spec(D):
    return pl.BlockSpec((BR, D), lambda i: (i, 0))


def _split_spec(DH):
    return pl.BlockSpec((NC, BR, DH), lambda i: (0, i, 0))


_deg_spec = pl.BlockSpec((1, NC, BR), lambda i: (i, 0, 0))


def _full_spec(shape):
    return pl.BlockSpec(shape, lambda i: tuple(0 for _ in shape))


def kernel(x, edge_index, W1, W2):
    npad = E_PAD - E
    src3d = jnp.concatenate(
        [edge_index[0], jnp.zeros((npad,), edge_index.dtype)]
    ).reshape(NS, CPS, CHUNK)
    dst3d = jnp.concatenate(
        [edge_index[1], jnp.full((npad,), N, edge_index.dtype)]
    ).reshape(NS, CPS, CHUNK)
    dst3d_w = dst3d.reshape(NW, CPS // NC, CHUNK)

    degp0, degp1 = _deg_kernel(dst3d_w)
    degp = jnp.stack([degp0.reshape(GRID, BR), degp1.reshape(GRID, BR)],
                     axis=1)

    h1 = pl.pallas_call(
        _mm1_body,
        grid=(GRID,),
        in_specs=[_rows_spec(D_IN), _full_spec((D_IN, H1))],
        out_specs=_rows_spec(H1),
        out_shape=jax.ShapeDtypeStruct((N, H1), jnp.float32),
    )(x, W1)

    h1p = pl.pallas_call(
        _scale_split_body,
        grid=(GRID,),
        in_specs=[_rows_spec(H1), _deg_spec],
        out_specs=_split_spec(H1 // NC),
        out_shape=jax.ShapeDtypeStruct((NC, N, H1 // NC), jnp.float32),
    )(h1, degp)

    q = _agg64(h1p, src3d, dst3d)

    h2p = pl.pallas_call(
        _layer2_body,
        grid=(GRID,),
        in_specs=[_split_spec(H1 // NC), _split_spec(H1 // NC),
                  _deg_spec, _full_spec((H1, H2))],
        out_specs=_split_spec(H2 // NC),
        out_shape=jax.ShapeDtypeStruct((NC, N, H2 // NC), jnp.float32),
    )(q, h1p, degp, W2)

    r = _agg32(h2p, src3d, dst3d)

    z = pl.pallas_call(
        _final_body,
        grid=(GRID,),
        in_specs=[_split_spec(H2 // NC), _split_spec(H2 // NC),
                  _deg_spec],
        out_specs=_rows_spec(H2),
        out_shape=jax.ShapeDtypeStruct((N, H2), jnp.float32),
    )(r, h2p, degp)

    return z


# speedup vs baseline: 1.0043x; 1.0043x over previous
import functools

import jax
import jax.numpy as jnp
from jax import lax
from jax.experimental import pallas as pl
from jax.experimental.pallas import tpu as pltpu
from jax.experimental.pallas import tpu_sc as plsc

N = 10000
E = 320000
D_IN = 128
H1 = 128
H2 = 64

NC = 2
NS = 16
NW = NC * NS
CHUNK = 128
CPS = 160
E_PAD = NS * CPS * CHUNK
NPAD = N + 8
NBUF = 4
ABUF = 2
AG = 2
_DIAG = 0
STRIPE_A = 632
STRIPE_B = N - 15 * STRIPE_A

_mesh = plsc.VectorSubcoreMesh(core_axis_name="c", subcore_axis_name="s")
_sc_params = pltpu.CompilerParams(use_tc_tiling_on_sc=False)


def _stripe_sizes(tile15):
    total = STRIPE_B if tile15 else STRIPE_A
    sizes = []
    while total:
        k = min(CHUNK, total)
        sizes.append(k)
        total -= k
    return sizes


def _stripe_pieces(piece_fn, s):
    @pl.when(s < 15)
    def _():
        off = 0
        for k in _stripe_sizes(False):
            piece_fn(s * STRIPE_A + off, k)
            off += k
    @pl.when(s == 15)
    def _():
        off = 0
        for k in _stripe_sizes(True):
            piece_fn(15 * STRIPE_A + off, k)
            off += k


def _make_deg_kernel():
    @functools.partial(
        pl.kernel,
        out_type=(jax.ShapeDtypeStruct((N,), jnp.float32),
                  jax.ShapeDtypeStruct((N,), jnp.float32)),
        mesh=_mesh,
        compiler_params=_sc_params,
        scratch_types=[
            pltpu.VMEM((CPS // NC, CHUNK), jnp.int32),
            pltpu.VMEM((CHUNK,), jnp.float32),
            pltpu.VMEM((640,), jnp.float32),
            pltpu.VMEM_SHARED((NPAD,), jnp.float32),
            pltpu.SemaphoreType.DMA,
            pltpu.SemaphoreType.DMA,
            pltpu.SemaphoreType.DMA,
            pltpu.SemaphoreType.DMA,
            pltpu.SemaphoreType.DMA,
        ],
    )
    def deg_kernel(dst3d, out0, out1, didx, ones, zbuf, acc,
                   isem, s0, s1, s2, s3):
        ssem = [s0, s1, s2, s3]
        c = lax.axis_index("c")
        s = lax.axis_index("s")
        wid = s * NC + c

        idesc = pltpu.async_copy(dst3d.at[wid], didx, isem)
        for g in range(8):
            ones[pl.ds(g * 16, 16)] = jnp.ones((16,), jnp.float32)
        for g in range(40):
            zbuf[pl.ds(g * 16, 16)] = jnp.zeros((16,), jnp.float32)
        _stripe_pieces(lambda o, n: pltpu.sync_copy(
            zbuf.at[pl.ds(0, n)], acc.at[pl.ds(o, n)]), s)
        idesc.wait()
        plsc.subcore_barrier()

        def group(g, carry):
            descs = []
            for b in range(NBUF):
                descs.append(pltpu.async_copy(
                    ones, acc.at[didx.at[g * NBUF + b]], ssem[b], add=True))
            for b in range(NBUF):
                descs[b].wait()
            return carry
        lax.fori_loop(0, CPS // NC // NBUF, group, 0)
        plsc.subcore_barrier()

        def bounce(outref):
            def piece(o, n):
                pltpu.sync_copy(acc.at[pl.ds(o, n)], zbuf.at[pl.ds(0, n)])
                pltpu.sync_copy(zbuf.at[pl.ds(0, n)], outref.at[pl.ds(o, n)])
            return piece
        @pl.when(c == 0)
        def _():
            _stripe_pieces(bounce(out0), s)
        @pl.when(c == 1)
        def _():
            _stripe_pieces(bounce(out1), s)

    return deg_kernel


def _make_agg_kernel(DH):
    @functools.partial(
        pl.kernel,
        out_type=jax.ShapeDtypeStruct((NC, N, DH), jnp.float32),
        mesh=_mesh,
        compiler_params=_sc_params,
        scratch_types=[
            pltpu.VMEM((CPS * CHUNK,), jnp.int32),
            pltpu.VMEM((CPS * CHUNK,), jnp.int32),
            pltpu.VMEM((ABUF, AG * CHUNK, DH), jnp.float32),
            pltpu.VMEM_SHARED((NPAD, DH), jnp.float32),
            pltpu.SemaphoreType.DMA,
        ] + [pltpu.SemaphoreType.DMA] * (2 * ABUF),
    )
    def agg_kernel(h_split, src2d, dst2d, out,
                   sidx, didx, buf, acc, isem, *sems):
        gsem = list(sems[:ABUF])
        ssem = list(sems[ABUF:])
        c = lax.axis_index("c")
        s = lax.axis_index("s")
        h_c = h_split.at[c]

        d1 = pltpu.async_copy(src2d.at[s], sidx, isem)
        d2 = pltpu.async_copy(dst2d.at[s], didx, isem)
        def zrow(i, carry):
            for g in range(DH // 16):
                buf[0, i, pl.ds(g * 16, 16)] = jnp.zeros((16,), jnp.float32)
            return carry
        lax.fori_loop(0, CHUNK, zrow, 0)
        _stripe_pieces(lambda o, n: pltpu.sync_copy(
            buf.at[0, pl.ds(0, n)], acc.at[pl.ds(o, n)]), s)
        d1.wait()
        d2.wait()
        plsc.subcore_barrier()

        def fire_scatter(b, j):
            if _DIAG == 1:
                pltpu.async_copy(
                    buf.at[b], acc.at[pl.ds(0, AG * CHUNK)], ssem[b])
            else:
                pltpu.async_copy(
                    buf.at[b], acc.at[didx.at[pl.ds(j * AG * CHUNK,
                                                    AG * CHUNK)]], ssem[b],
                    add=True)

        def drain_scatter(b, j):
            if _DIAG == 1:
                pltpu.make_async_copy(
                    buf.at[b], acc.at[pl.ds(0, AG * CHUNK)], ssem[b]).wait()
            else:
                pltpu.make_async_copy(
                    buf.at[b], acc.at[didx.at[pl.ds(j * AG * CHUNK,
                                                    AG * CHUNK)]],
                    ssem[b]).wait()

        def fire_gather(b, j):
            if _DIAG == 2:
                return pltpu.async_copy(
                    h_c.at[pl.ds(0, AG * CHUNK)], buf.at[b], gsem[b])
            return pltpu.async_copy(
                h_c.at[sidx.at[pl.ds(j * AG * CHUNK, AG * CHUNK)]],
                buf.at[b], gsem[b])

        def group(g, carry):
            gd = []
            for b in range(ABUF):
                @pl.when(g > 0)
                def _(b=b):
                    drain_scatter(b, (g - 1) * ABUF + b)
                gd.append(fire_gather(b, g * ABUF + b))
            for b in range(ABUF):
                gd[b].wait()
                fire_scatter(b, g * ABUF + b)
            return carry
        ng = CPS // AG // ABUF
        lax.fori_loop(0, ng, group, 0)
        for b in range(ABUF):
            drain_scatter(b, (ng - 1) * ABUF + b)
        plsc.subcore_barrier()

        def piece(o, n):
            pltpu.sync_copy(acc.at[pl.ds(o, n)], buf.at[0, pl.ds(0, n)])
            pltpu.sync_copy(buf.at[0, pl.ds(0, n)], out.at[c, pl.ds(o, n)])
        _stripe_pieces(piece, s)

    return agg_kernel


_deg_kernel = _make_deg_kernel()
_agg64 = _make_agg_kernel(H1 // NC)
_agg32 = _make_agg_kernel(H2 // NC)

BR = 1000
GRID = N // BR


def _dinv_block(degp_ref):
    p = degp_ref[...]
    return lax.rsqrt(1.0 + p[0, 0] + p[0, 1])


def _mm1_body(x_ref, w_ref, o_ref):
    o_ref[...] = jnp.dot(x_ref[...], w_ref[...],
                         preferred_element_type=jnp.float32)


def _scale_split_body(h_ref, degp_ref, o_ref):
    dinv = _dinv_block(degp_ref)
    hp = h_ref[...] * dinv[:, None]
    d = hp.shape[1] // 2
    o_ref[0] = hp[:, :d]
    o_ref[1] = hp[:, d:]


def _layer2_body(q_ref, hp_ref, degp_ref, w_ref, o_ref):
    dinv = _dinv_block(degp_ref)
    agg = jnp.concatenate([q_ref[0] + hp_ref[0], q_ref[1] + hp_ref[1]],
                          axis=1)
    z1 = jnp.maximum(agg * dinv[:, None], 0.0)
    h2 = jnp.dot(z1, w_ref[...], preferred_element_type=jnp.float32)
    hp2 = h2 * dinv[:, None]
    d = hp2.shape[1] // 2
    o_ref[0] = hp2[:, :d]
    o_ref[1] = hp2[:, d:]


def _final_body(r_ref, hp_ref, degp_ref, o_ref):
    dinv = _dinv_block(degp_ref)
    agg = jnp.concatenate([r_ref[0] + hp_ref[0], r_ref[1] + hp_ref[1]],
                          axis=1)
    o_ref[...] = agg * dinv[:, None]


def _rows_spec(D):
    return pl.BlockSpec((BR, D), lambda i: (i, 0))


def _split_spec(DH):
    return pl.BlockSpec((NC, BR, DH), lambda i: (0, i, 0))


_deg_spec = pl.BlockSpec((1, NC, BR), lambda i: (i, 0, 0))


def _full_spec(shape):
    return pl.BlockSpec(shape, lambda i: tuple(0 for _ in shape))


def kernel(x, edge_index, W1, W2):
    npad = E_PAD - E
    src2d = jnp.concatenate(
        [edge_index[0], jnp.zeros((npad,), edge_index.dtype)]
    ).reshape(NS, CPS * CHUNK)
    dst2d = jnp.concatenate(
        [edge_index[1], jnp.full((npad,), N, edge_index.dtype)]
    ).reshape(NS, CPS * CHUNK)
    dst3d_w = dst2d.reshape(NW, CPS // NC, CHUNK)

    degp0, degp1 = _deg_kernel(dst3d_w)
    degp = jnp.stack([degp0.reshape(GRID, BR), degp1.reshape(GRID, BR)],
                     axis=1)

    h1 = pl.pallas_call(
        _mm1_body,
        grid=(GRID,),
        in_specs=[_rows_spec(D_IN), _full_spec((D_IN, H1))],
        out_specs=_rows_spec(H1),
        out_shape=jax.ShapeDtypeStruct((N, H1), jnp.float32),
    )(x, W1)

    h1p = pl.pallas_call(
        _scale_split_body,
        grid=(GRID,),
        in_specs=[_rows_spec(H1), _deg_spec],
        out_specs=_split_spec(H1 // NC),
        out_shape=jax.ShapeDtypeStruct((NC, N, H1 // NC), jnp.float32),
    )(h1, degp)

    q = _agg64(h1p, src2d, dst2d)

    h2p = pl.pallas_call(
        _layer2_body,
        grid=(GRID,),
        in_specs=[_split_spec(H1 // NC), _split_spec(H1 // NC),
                  _deg_spec, _full_spec((H1, H2))],
        out_specs=_split_spec(H2 // NC),
        out_shape=jax.ShapeDtypeStruct((NC, N, H2 // NC), jnp.float32),
    )(q, h1p, degp, W2)

    r = _agg32(h2p, src2d, dst2d)

    z = pl.pallas_call(
        _final_body,
        grid=(GRID,),
        in_specs=[_split_spec(H2 // NC), _split_spec(H2 // NC),
                  _deg_spec],
        out_specs=_rows_spec(H2),
        out_shape=jax.ShapeDtypeStruct((N, H2), jnp.float32),
    )(r, h2p, degp)

    return z


# speedup vs baseline: 1.0573x; 1.0528x over previous
import functools

import jax
import jax.numpy as jnp
from jax import lax
from jax.experimental import pallas as pl
from jax.experimental.pallas import tpu as pltpu
from jax.experimental.pallas import tpu_sc as plsc

N = 10000
E = 320000
D_IN = 128
H1 = 128
H2 = 64

NC = 2
NS = 16
NW = NC * NS
CHUNK = 128
CPS = 160
E_PAD = NS * CPS * CHUNK
NPAD = N + 8
NBUF = 4
ABUF = 5
AG = 1
_DIAG = 0
STRIPE_A = 632
STRIPE_B = N - 15 * STRIPE_A

_mesh = plsc.VectorSubcoreMesh(core_axis_name="c", subcore_axis_name="s")
_sc_params = pltpu.CompilerParams(use_tc_tiling_on_sc=False)


def _stripe_sizes(tile15):
    total = STRIPE_B if tile15 else STRIPE_A
    sizes = []
    while total:
        k = min(CHUNK, total)
        sizes.append(k)
        total -= k
    return sizes


def _stripe_pieces(piece_fn, s):
    @pl.when(s < 15)
    def _():
        off = 0
        for k in _stripe_sizes(False):
            piece_fn(s * STRIPE_A + off, k)
            off += k
    @pl.when(s == 15)
    def _():
        off = 0
        for k in _stripe_sizes(True):
            piece_fn(15 * STRIPE_A + off, k)
            off += k


def _make_deg_kernel():
    @functools.partial(
        pl.kernel,
        out_type=(jax.ShapeDtypeStruct((N,), jnp.float32),
                  jax.ShapeDtypeStruct((N,), jnp.float32)),
        mesh=_mesh,
        compiler_params=_sc_params,
        scratch_types=[
            pltpu.VMEM((CPS // NC, CHUNK), jnp.int32),
            pltpu.VMEM((CHUNK,), jnp.float32),
            pltpu.VMEM((640,), jnp.float32),
            pltpu.VMEM_SHARED((NPAD,), jnp.float32),
            pltpu.SemaphoreType.DMA,
            pltpu.SemaphoreType.DMA,
            pltpu.SemaphoreType.DMA,
            pltpu.SemaphoreType.DMA,
            pltpu.SemaphoreType.DMA,
        ],
    )
    def deg_kernel(dst3d, out0, out1, didx, ones, zbuf, acc,
                   isem, s0, s1, s2, s3):
        ssem = [s0, s1, s2, s3]
        c = lax.axis_index("c")
        s = lax.axis_index("s")
        wid = s * NC + c

        idesc = pltpu.async_copy(dst3d.at[wid], didx, isem)
        for g in range(8):
            ones[pl.ds(g * 16, 16)] = jnp.ones((16,), jnp.float32)
        for g in range(40):
            zbuf[pl.ds(g * 16, 16)] = jnp.zeros((16,), jnp.float32)
        _stripe_pieces(lambda o, n: pltpu.sync_copy(
            zbuf.at[pl.ds(0, n)], acc.at[pl.ds(o, n)]), s)
        idesc.wait()
        plsc.subcore_barrier()

        def group(g, carry):
            descs = []
            for b in range(NBUF):
                descs.append(pltpu.async_copy(
                    ones, acc.at[didx.at[g * NBUF + b]], ssem[b], add=True))
            for b in range(NBUF):
                descs[b].wait()
            return carry
        lax.fori_loop(0, CPS // NC // NBUF, group, 0)
        plsc.subcore_barrier()

        def bounce(outref):
            def piece(o, n):
                pltpu.sync_copy(acc.at[pl.ds(o, n)], zbuf.at[pl.ds(0, n)])
                pltpu.sync_copy(zbuf.at[pl.ds(0, n)], outref.at[pl.ds(o, n)])
            return piece
        @pl.when(c == 0)
        def _():
            _stripe_pieces(bounce(out0), s)
        @pl.when(c == 1)
        def _():
            _stripe_pieces(bounce(out1), s)

    return deg_kernel


def _make_agg_kernel(DH):
    @functools.partial(
        pl.kernel,
        out_type=jax.ShapeDtypeStruct((NC, N, DH), jnp.float32),
        mesh=_mesh,
        compiler_params=_sc_params,
        scratch_types=[
            pltpu.VMEM((CPS * CHUNK,), jnp.int32),
            pltpu.VMEM((CPS * CHUNK,), jnp.int32),
            pltpu.VMEM((ABUF, AG * CHUNK, DH), jnp.float32),
            pltpu.VMEM_SHARED((NPAD, DH), jnp.float32),
            pltpu.SemaphoreType.DMA,
        ] + [pltpu.SemaphoreType.DMA] * (2 * ABUF),
    )
    def agg_kernel(h_split, src2d, dst2d, out,
                   sidx, didx, buf, acc, isem, *sems):
        gsem = list(sems[:ABUF])
        ssem = list(sems[ABUF:])
        c = lax.axis_index("c")
        s = lax.axis_index("s")
        h_c = h_split.at[c]

        d1 = pltpu.async_copy(src2d.at[s], sidx, isem)
        d2 = pltpu.async_copy(dst2d.at[s], didx, isem)
        def zrow(i, carry):
            for g in range(DH // 16):
                buf[0, i, pl.ds(g * 16, 16)] = jnp.zeros((16,), jnp.float32)
            return carry
        lax.fori_loop(0, CHUNK, zrow, 0)
        _stripe_pieces(lambda o, n: pltpu.sync_copy(
            buf.at[0, pl.ds(0, n)], acc.at[pl.ds(o, n)]), s)
        d1.wait()
        d2.wait()
        plsc.subcore_barrier()

        def fire_scatter(b, j):
            if _DIAG == 1:
                pltpu.async_copy(
                    buf.at[b], acc.at[pl.ds(0, AG * CHUNK)], ssem[b])
            else:
                pltpu.async_copy(
                    buf.at[b], acc.at[didx.at[pl.ds(j * AG * CHUNK,
                                                    AG * CHUNK)]], ssem[b],
                    add=True)

        def drain_scatter(b, j):
            if _DIAG == 1:
                pltpu.make_async_copy(
                    buf.at[b], acc.at[pl.ds(0, AG * CHUNK)], ssem[b]).wait()
            else:
                pltpu.make_async_copy(
                    buf.at[b], acc.at[didx.at[pl.ds(j * AG * CHUNK,
                                                    AG * CHUNK)]],
                    ssem[b]).wait()

        def fire_gather(b, j):
            if _DIAG == 2:
                return pltpu.async_copy(
                    h_c.at[pl.ds(0, AG * CHUNK)], buf.at[b], gsem[b])
            return pltpu.async_copy(
                h_c.at[sidx.at[pl.ds(j * AG * CHUNK, AG * CHUNK)]],
                buf.at[b], gsem[b])

        def group(g, carry):
            gd = []
            for b in range(ABUF):
                @pl.when(g > 0)
                def _(b=b):
                    drain_scatter(b, (g - 1) * ABUF + b)
                gd.append(fire_gather(b, g * ABUF + b))
            for b in range(ABUF):
                gd[b].wait()
                fire_scatter(b, g * ABUF + b)
            return carry
        ng = CPS // AG // ABUF
        lax.fori_loop(0, ng, group, 0)
        for b in range(ABUF):
            drain_scatter(b, (ng - 1) * ABUF + b)
        plsc.subcore_barrier()

        def piece(o, n):
            pltpu.sync_copy(acc.at[pl.ds(o, n)], buf.at[0, pl.ds(0, n)])
            pltpu.sync_copy(buf.at[0, pl.ds(0, n)], out.at[c, pl.ds(o, n)])
        _stripe_pieces(piece, s)

    return agg_kernel


_deg_kernel = _make_deg_kernel()
_agg64 = _make_agg_kernel(H1 // NC)
_agg32 = _make_agg_kernel(H2 // NC)

BR = 1000
GRID = N // BR


def _dinv_block(degp_ref):
    p = degp_ref[...]
    return lax.rsqrt(1.0 + p[0, 0] + p[0, 1])


def _mm1_body(x_ref, w_ref, o_ref):
    o_ref[...] = jnp.dot(x_ref[...], w_ref[...],
                         preferred_element_type=jnp.float32)


def _scale_split_body(h_ref, degp_ref, o_ref):
    dinv = _dinv_block(degp_ref)
    hp = h_ref[...] * dinv[:, None]
    d = hp.shape[1] // 2
    o_ref[0] = hp[:, :d]
    o_ref[1] = hp[:, d:]


def _layer2_body(q_ref, hp_ref, degp_ref, w_ref, o_ref):
    dinv = _dinv_block(degp_ref)
    agg = jnp.concatenate([q_ref[0] + hp_ref[0], q_ref[1] + hp_ref[1]],
                          axis=1)
    z1 = jnp.maximum(agg * dinv[:, None], 0.0)
    h2 = jnp.dot(z1, w_ref[...], preferred_element_type=jnp.float32)
    hp2 = h2 * dinv[:, None]
    d = hp2.shape[1] // 2
    o_ref[0] = hp2[:, :d]
    o_ref[1] = hp2[:, d:]


def _final_body(r_ref, hp_ref, degp_ref, o_ref):
    dinv = _dinv_block(degp_ref)
    agg = jnp.concatenate([r_ref[0] + hp_ref[0], r_ref[1] + hp_ref[1]],
                          axis=1)
    o_ref[...] = agg * dinv[:, None]


def _rows_spec(D):
    return pl.BlockSpec((BR, D), lambda i: (i, 0))


def _split_spec(DH):
    return pl.BlockSpec((NC, BR, DH), lambda i: (0, i, 0))


_deg_spec = pl.BlockSpec((1, NC, BR), lambda i: (i, 0, 0))


def _full_spec(shape):
    return pl.BlockSpec(shape, lambda i: tuple(0 for _ in shape))


def kernel(x, edge_index, W1, W2):
    npad = E_PAD - E
    src2d = jnp.concatenate(
        [edge_index[0], jnp.zeros((npad,), edge_index.dtype)]
    ).reshape(NS, CPS * CHUNK)
    dst2d = jnp.concatenate(
        [edge_index[1], jnp.full((npad,), N, edge_index.dtype)]
    ).reshape(NS, CPS * CHUNK)
    dst3d_w = dst2d.reshape(NW, CPS // NC, CHUNK)

    degp0, degp1 = _deg_kernel(dst3d_w)
    degp = jnp.stack([degp0.reshape(GRID, BR), degp1.reshape(GRID, BR)],
                     axis=1)

    h1 = pl.pallas_call(
        _mm1_body,
        grid=(GRID,),
        in_specs=[_rows_spec(D_IN), _full_spec((D_IN, H1))],
        out_specs=_rows_spec(H1),
        out_shape=jax.ShapeDtypeStruct((N, H1), jnp.float32),
    )(x, W1)

    h1p = pl.pallas_call(
        _scale_split_body,
        grid=(GRID,),
        in_specs=[_rows_spec(H1), _deg_spec],
        out_specs=_split_spec(H1 // NC),
        out_shape=jax.ShapeDtypeStruct((NC, N, H1 // NC), jnp.float32),
    )(h1, degp)

    q = _agg64(h1p, src2d, dst2d)

    h2p = pl.pallas_call(
        _layer2_body,
        grid=(GRID,),
        in_specs=[_split_spec(H1 // NC), _split_spec(H1 // NC),
                  _deg_spec, _full_spec((H1, H2))],
        out_specs=_split_spec(H2 // NC),
        out_shape=jax.ShapeDtypeStruct((NC, N, H2 // NC), jnp.float32),
    )(q, h1p, degp, W2)

    r = _agg32(h2p, src2d, dst2d)

    z = pl.pallas_call(
        _final_body,
        grid=(GRID,),
        in_specs=[_split_spec(H2 // NC), _split_spec(H2 // NC),
                  _deg_spec],
        out_specs=_rows_spec(H2),
        out_shape=jax.ShapeDtypeStruct((N, H2), jnp.float32),
    )(r, h2p, degp)

    return z


# speedup vs baseline: 1.2555x; 1.1874x over previous
import functools

import jax
import jax.numpy as jnp
from jax import lax
from jax.experimental import pallas as pl
from jax.experimental.pallas import tpu as pltpu
from jax.experimental.pallas import tpu_sc as plsc

N = 10000
E = 320000
D_IN = 128
H1 = 128
H2 = 64

NC = 2
NS = 16
NW = NC * NS
CHUNK = 128
CPS = 160
E_PAD = NS * CPS * CHUNK
NPAD = N + 8
NBUF = 4
ABUF = 4
STRIPE_A = 632
STRIPE_B = N - 15 * STRIPE_A

_mesh = plsc.VectorSubcoreMesh(core_axis_name="c", subcore_axis_name="s")
_sc_params = pltpu.CompilerParams(use_tc_tiling_on_sc=False)


def _stripe_sizes(tile15):
    total = STRIPE_B if tile15 else STRIPE_A
    sizes = []
    while total:
        k = min(CHUNK, total)
        sizes.append(k)
        total -= k
    return sizes


def _stripe_pieces(piece_fn, s):
    @pl.when(s < 15)
    def _():
        off = 0
        for k in _stripe_sizes(False):
            piece_fn(s * STRIPE_A + off, k)
            off += k
    @pl.when(s == 15)
    def _():
        off = 0
        for k in _stripe_sizes(True):
            piece_fn(15 * STRIPE_A + off, k)
            off += k


def _make_deg_kernel():
    @functools.partial(
        pl.kernel,
        out_type=(jax.ShapeDtypeStruct((N,), jnp.float32),
                  jax.ShapeDtypeStruct((N,), jnp.float32)),
        mesh=_mesh,
        compiler_params=_sc_params,
        scratch_types=[
            pltpu.VMEM((CPS // NC, CHUNK), jnp.int32),
            pltpu.VMEM((CHUNK,), jnp.float32),
            pltpu.VMEM((640,), jnp.float32),
            pltpu.VMEM_SHARED((NPAD,), jnp.float32),
            pltpu.SemaphoreType.DMA,
            pltpu.SemaphoreType.DMA,
            pltpu.SemaphoreType.DMA,
            pltpu.SemaphoreType.DMA,
            pltpu.SemaphoreType.DMA,
        ],
    )
    def deg_kernel(dst3d, out0, out1, didx, ones, zbuf, acc,
                   isem, s0, s1, s2, s3):
        ssem = [s0, s1, s2, s3]
        c = lax.axis_index("c")
        s = lax.axis_index("s")
        wid = s * NC + c

        idesc = pltpu.async_copy(dst3d.at[wid], didx, isem)
        for g in range(8):
            ones[pl.ds(g * 16, 16)] = jnp.ones((16,), jnp.float32)
        for g in range(40):
            zbuf[pl.ds(g * 16, 16)] = jnp.zeros((16,), jnp.float32)
        _stripe_pieces(lambda o, n: pltpu.sync_copy(
            zbuf.at[pl.ds(0, n)], acc.at[pl.ds(o, n)]), s)
        idesc.wait()
        plsc.subcore_barrier()

        def group(g, carry):
            descs = []
            for b in range(NBUF):
                descs.append(pltpu.async_copy(
                    ones, acc.at[didx.at[g * NBUF + b]], ssem[b], add=True))
            for b in range(NBUF):
                descs[b].wait()
            return carry
        lax.fori_loop(0, CPS // NC // NBUF, group, 0)
        plsc.subcore_barrier()

        def bounce(outref):
            def piece(o, n):
                pltpu.sync_copy(acc.at[pl.ds(o, n)], zbuf.at[pl.ds(0, n)])
                pltpu.sync_copy(zbuf.at[pl.ds(0, n)], outref.at[pl.ds(o, n)])
            return piece
        @pl.when(c == 0)
        def _():
            _stripe_pieces(bounce(out0), s)
        @pl.when(c == 1)
        def _():
            _stripe_pieces(bounce(out1), s)

    return deg_kernel


def _make_agg_kernel(DH):
    @functools.partial(
        pl.kernel,
        out_type=jax.ShapeDtypeStruct((NC, N, DH), jnp.float32),
        mesh=_mesh,
        compiler_params=_sc_params,
        scratch_types=[
            pltpu.VMEM((CPS * CHUNK,), jnp.int32),
            pltpu.VMEM((CPS * CHUNK,), jnp.int32),
            pltpu.VMEM((ABUF, CHUNK, DH // 2), jnp.int32),
            pltpu.VMEM((ABUF, CHUNK, DH), jnp.float32),
            pltpu.VMEM_SHARED((NPAD, DH), jnp.float32),
            pltpu.SemaphoreType.DMA,
        ] + [pltpu.SemaphoreType.DMA] * (2 * ABUF),
    )
    def agg_kernel(h_split, src2d, dst2d, out,
                   sidx, didx, gbuf, fbuf, acc, isem, *sems):
        gsem = list(sems[:ABUF])
        ssem = list(sems[ABUF:])
        c = lax.axis_index("c")
        s = lax.axis_index("s")
        h_c = h_split.at[c]

        d1 = pltpu.async_copy(src2d.at[s], sidx, isem)
        d2 = pltpu.async_copy(dst2d.at[s], didx, isem)
        def zrow(i, carry):
            for g in range(DH // 16):
                fbuf[0, i, pl.ds(g * 16, 16)] = jnp.zeros((16,), jnp.float32)
            return carry
        lax.fori_loop(0, CHUNK, zrow, 0)
        _stripe_pieces(lambda o, n: pltpu.sync_copy(
            fbuf.at[0, pl.ds(0, n)], acc.at[pl.ds(o, n)]), s)
        d1.wait()
        d2.wait()
        plsc.subcore_barrier()

        def fire_gather(b, j):
            return pltpu.async_copy(
                h_c.at[sidx.at[pl.ds(j * CHUNK, CHUNK)]], gbuf.at[b],
                gsem[b])

        def fire_scatter(b, j):
            pltpu.async_copy(
                fbuf.at[b], acc.at[didx.at[pl.ds(j * CHUNK, CHUNK)]],
                ssem[b], add=True)

        def drain_scatter(b, j):
            pltpu.make_async_copy(
                fbuf.at[b], acc.at[didx.at[pl.ds(j * CHUNK, CHUNK)]],
                ssem[b]).wait()

        def convert(b):
            def crow(r, carry):
                for g in range(DH // 32):
                    x = gbuf[b, r, pl.ds(16 * g, 16)]
                    lo = lax.bitcast_convert_type(x << 16, jnp.float32)
                    hi = lax.bitcast_convert_type(
                        x & jnp.int32(-65536), jnp.float32)
                    fbuf[b, r, pl.ds(32 * g, 16)] = lo
                    fbuf[b, r, pl.ds(32 * g + 16, 16)] = hi
                return carry
            lax.fori_loop(0, CHUNK, crow, 0)

        gd = []
        for b in range(ABUF):
            gd.append(fire_gather(b, b))
        ng = CPS // ABUF

        def group(g, carry):
            for b in range(ABUF):
                j = g * ABUF + b
                @pl.when(g > 0)
                def _(b=b, j=j):
                    drain_scatter(b, j - ABUF)
                pltpu.make_async_copy(
                    h_c.at[sidx.at[pl.ds(j * CHUNK, CHUNK)]], gbuf.at[b],
                    gsem[b]).wait()
                convert(b)
                fire_scatter(b, j)
                @pl.when(g < ng - 1)
                def _(b=b, j=j):
                    fire_gather(b, j + ABUF)
            return carry
        lax.fori_loop(0, ng, group, 0)
        for b in range(ABUF):
            drain_scatter(b, (ng - 1) * ABUF + b)
        plsc.subcore_barrier()

        def piece(o, n):
            pltpu.sync_copy(acc.at[pl.ds(o, n)], fbuf.at[0, pl.ds(0, n)])
            pltpu.sync_copy(fbuf.at[0, pl.ds(0, n)], out.at[c, pl.ds(o, n)])
        _stripe_pieces(piece, s)

    return agg_kernel


_deg_kernel = _make_deg_kernel()
_agg64 = _make_agg_kernel(H1 // NC)
_agg32 = _make_agg_kernel(H2 // NC)

BR = 1000
GRID = N // BR


def _dinv_block(degp_ref):
    p = degp_ref[...]
    return lax.rsqrt(1.0 + p[0, 0] + p[0, 1])


def _mm1_body(x_ref, w_ref, o_ref):
    o_ref[...] = jnp.dot(x_ref[...], w_ref[...],
                         preferred_element_type=jnp.float32)


def _scale_split_body(h_ref, degp_ref, o_ref):
    dinv = _dinv_block(degp_ref)
    hp = h_ref[...] * dinv[:, None]
    d = hp.shape[1] // 2
    o_ref[0] = hp[:, :d]
    o_ref[1] = hp[:, d:]


def _layer2_body(q_ref, hp_ref, degp_ref, w_ref, o_ref):
    dinv = _dinv_block(degp_ref)
    agg = jnp.concatenate([q_ref[0] + hp_ref[0], q_ref[1] + hp_ref[1]],
                          axis=1)
    z1 = jnp.maximum(agg * dinv[:, None], 0.0)
    h2 = jnp.dot(z1, w_ref[...], preferred_element_type=jnp.float32)
    hp2 = h2 * dinv[:, None]
    d = hp2.shape[1] // 2
    o_ref[0] = hp2[:, :d]
    o_ref[1] = hp2[:, d:]


def _final_body(r_ref, hp_ref, degp_ref, o_ref):
    dinv = _dinv_block(degp_ref)
    agg = jnp.concatenate([r_ref[0] + hp_ref[0], r_ref[1] + hp_ref[1]],
                          axis=1)
    o_ref[...] = agg * dinv[:, None]


def _bf16_perm(hp):
    dh = hp.shape[-1]
    b = (hp.astype(jnp.bfloat16)
         .reshape(NC, N, dh // 32, 2, 16)
         .transpose(0, 1, 2, 4, 3))
    return lax.bitcast_convert_type(b, jnp.int32).reshape(NC, N, dh // 2)


def _rows_spec(D):
    return pl.BlockSpec((BR, D), lambda i: (i, 0))


def _split_spec(DH):
    return pl.BlockSpec((NC, BR, DH), lambda i: (0, i, 0))


_deg_spec = pl.BlockSpec((1, NC, BR), lambda i: (i, 0, 0))


def _full_spec(shape):
    return pl.BlockSpec(shape, lambda i: tuple(0 for _ in shape))


def kernel(x, edge_index, W1, W2):
    npad = E_PAD - E
    src2d = jnp.concatenate(
        [edge_index[0], jnp.zeros((npad,), edge_index.dtype)]
    ).reshape(NS, CPS * CHUNK)
    dst2d = jnp.concatenate(
        [edge_index[1], jnp.full((npad,), N, edge_index.dtype)]
    ).reshape(NS, CPS * CHUNK)
    dst3d_w = dst2d.reshape(NW, CPS // NC, CHUNK)

    degp0, degp1 = _deg_kernel(dst3d_w)
    degp = jnp.stack([degp0.reshape(GRID, BR), degp1.reshape(GRID, BR)],
                     axis=1)

    h1 = pl.pallas_call(
        _mm1_body,
        grid=(GRID,),
        in_specs=[_rows_spec(D_IN), _full_spec((D_IN, H1))],
        out_specs=_rows_spec(H1),
        out_shape=jax.ShapeDtypeStruct((N, H1), jnp.float32),
    )(x, W1)

    h1p = pl.pallas_call(
        _scale_split_body,
        grid=(GRID,),
        in_specs=[_rows_spec(H1), _deg_spec],
        out_specs=_split_spec(H1 // NC),
        out_shape=jax.ShapeDtypeStruct((NC, N, H1 // NC), jnp.float32),
    )(h1, degp)

    q = _agg64(_bf16_perm(h1p), src2d, dst2d)

    h2p = pl.pallas_call(
        _layer2_body,
        grid=(GRID,),
        in_specs=[_split_spec(H1 // NC), _split_spec(H1 // NC),
                  _deg_spec, _full_spec((H1, H2))],
        out_specs=_split_spec(H2 // NC),
        out_shape=jax.ShapeDtypeStruct((NC, N, H2 // NC), jnp.float32),
    )(q, h1p, degp, W2)

    r = _agg32(_bf16_perm(h2p), src2d, dst2d)

    z = pl.pallas_call(
        _final_body,
        grid=(GRID,),
        in_specs=[_split_spec(H2 // NC), _split_spec(H2 // NC),
                  _deg_spec],
        out_specs=_rows_spec(H2),
        out_shape=jax.ShapeDtypeStruct((N, H2), jnp.float32),
    )(r, h2p, degp)

    return z


# speedup vs baseline: 1.3481x; 1.0737x over previous
import functools

import jax
import jax.numpy as jnp
from jax import lax
from jax.experimental import pallas as pl
from jax.experimental.pallas import tpu as pltpu
from jax.experimental.pallas import tpu_sc as plsc

N = 10000
E = 320000
D_IN = 128
H1 = 128
H2 = 64

NC = 2
NS = 16
NW = NC * NS
CHUNK = 128
CPS = 160
E_PAD = NS * CPS * CHUNK
NPAD = N + 8
NBUF = 4
ABUF = 4
STRIPE_A = 632
STRIPE_B = N - 15 * STRIPE_A

_mesh = plsc.VectorSubcoreMesh(core_axis_name="c", subcore_axis_name="s")
_sc_params = pltpu.CompilerParams(use_tc_tiling_on_sc=False)


def _stripe_sizes(tile15):
    total = STRIPE_B if tile15 else STRIPE_A
    sizes = []
    while total:
        k = min(CHUNK, total)
        sizes.append(k)
        total -= k
    return sizes


def _stripe_pieces(piece_fn, s):
    @pl.when(s < 15)
    def _():
        off = 0
        for k in _stripe_sizes(False):
            piece_fn(s * STRIPE_A + off, k)
            off += k
    @pl.when(s == 15)
    def _():
        off = 0
        for k in _stripe_sizes(True):
            piece_fn(15 * STRIPE_A + off, k)
            off += k


def _make_deg_kernel():
    @functools.partial(
        pl.kernel,
        out_type=(jax.ShapeDtypeStruct((N,), jnp.float32),
                  jax.ShapeDtypeStruct((N,), jnp.float32)),
        mesh=_mesh,
        compiler_params=_sc_params,
        scratch_types=[
            pltpu.VMEM((CPS // NC, CHUNK), jnp.int32),
            pltpu.VMEM((CHUNK,), jnp.float32),
            pltpu.VMEM((640,), jnp.float32),
            pltpu.VMEM_SHARED((NPAD,), jnp.float32),
            pltpu.SemaphoreType.DMA,
            pltpu.SemaphoreType.DMA,
            pltpu.SemaphoreType.DMA,
            pltpu.SemaphoreType.DMA,
            pltpu.SemaphoreType.DMA,
        ],
    )
    def deg_kernel(dst3d, out0, out1, didx, ones, zbuf, acc,
                   isem, s0, s1, s2, s3):
        ssem = [s0, s1, s2, s3]
        c = lax.axis_index("c")
        s = lax.axis_index("s")
        wid = s * NC + c

        idesc = pltpu.async_copy(dst3d.at[wid], didx, isem)
        for g in range(8):
            ones[pl.ds(g * 16, 16)] = jnp.ones((16,), jnp.float32)
        for g in range(40):
            zbuf[pl.ds(g * 16, 16)] = jnp.zeros((16,), jnp.float32)
        _stripe_pieces(lambda o, n: pltpu.sync_copy(
            zbuf.at[pl.ds(0, n)], acc.at[pl.ds(o, n)]), s)
        idesc.wait()
        plsc.subcore_barrier()

        def group(g, carry):
            descs = []
            for b in range(NBUF):
                descs.append(pltpu.async_copy(
                    ones, acc.at[didx.at[g * NBUF + b]], ssem[b], add=True))
            for b in range(NBUF):
                descs[b].wait()
            return carry
        lax.fori_loop(0, CPS // NC // NBUF, group, 0)
        plsc.subcore_barrier()

        def bounce(outref):
            def piece(o, n):
                pltpu.sync_copy(acc.at[pl.ds(o, n)], zbuf.at[pl.ds(0, n)])
                pltpu.sync_copy(zbuf.at[pl.ds(0, n)], outref.at[pl.ds(o, n)])
            return piece
        @pl.when(c == 0)
        def _():
            _stripe_pieces(bounce(out0), s)
        @pl.when(c == 1)
        def _():
            _stripe_pieces(bounce(out1), s)

    return deg_kernel


def _make_agg_kernel(DH):
    @functools.partial(
        pl.kernel,
        out_type=jax.ShapeDtypeStruct((NC, N, DH), jnp.float32),
        mesh=_mesh,
        compiler_params=_sc_params,
        scratch_types=[
            pltpu.VMEM((CPS * CHUNK,), jnp.int32),
            pltpu.VMEM((CPS * CHUNK,), jnp.int32),
            pltpu.VMEM((ABUF, CHUNK, DH // 2), jnp.int32),
            pltpu.VMEM((ABUF, CHUNK, DH), jnp.float32),
            pltpu.VMEM_SHARED((NPAD, DH), jnp.float32),
            pltpu.SemaphoreType.DMA,
        ] + [pltpu.SemaphoreType.DMA] * (2 * ABUF),
    )
    def agg_kernel(h_split, src2d, dst2d, out,
                   sidx, didx, gbuf, fbuf, acc, isem, *sems):
        gsem = list(sems[:ABUF])
        ssem = list(sems[ABUF:])
        c = lax.axis_index("c")
        s = lax.axis_index("s")
        h_c = h_split.at[c]

        d1 = pltpu.async_copy(src2d.at[s], sidx, isem)
        d2 = pltpu.async_copy(dst2d.at[s], didx, isem)
        def zrow(i, carry):
            for g in range(DH // 16):
                fbuf[0, i, pl.ds(g * 16, 16)] = jnp.zeros((16,), jnp.float32)
            return carry
        lax.fori_loop(0, CHUNK, zrow, 0)
        _stripe_pieces(lambda o, n: pltpu.sync_copy(
            fbuf.at[0, pl.ds(0, n)], acc.at[pl.ds(o, n)]), s)
        d1.wait()
        d2.wait()
        plsc.subcore_barrier()

        def fire_gather(b, j):
            return pltpu.async_copy(
                h_c.at[sidx.at[pl.ds(j * CHUNK, CHUNK)]], gbuf.at[b],
                gsem[b])

        def fire_scatter(b, j):
            pltpu.async_copy(
                fbuf.at[b], acc.at[didx.at[pl.ds(j * CHUNK, CHUNK)]],
                ssem[b], add=True)

        def drain_scatter(b, j):
            pltpu.make_async_copy(
                fbuf.at[b], acc.at[didx.at[pl.ds(j * CHUNK, CHUNK)]],
                ssem[b]).wait()

        def convert(b):
            def crow(r, carry):
                for g in range(DH // 32):
                    x = gbuf[b, r, pl.ds(16 * g, 16)]
                    lo = lax.bitcast_convert_type(x << 16, jnp.float32)
                    hi = lax.bitcast_convert_type(
                        x & jnp.int32(-65536), jnp.float32)
                    fbuf[b, r, pl.ds(16 * g, 16)] = lo
                    fbuf[b, r, pl.ds(DH // 2 + 16 * g, 16)] = hi
                return carry
            lax.fori_loop(0, CHUNK, crow, 0)

        gd = []
        for b in range(ABUF):
            gd.append(fire_gather(b, b))
        ng = CPS // ABUF

        def group(g, carry):
            for b in range(ABUF):
                j = g * ABUF + b
                @pl.when(g > 0)
                def _(b=b, j=j):
                    drain_scatter(b, j - ABUF)
                pltpu.make_async_copy(
                    h_c.at[sidx.at[pl.ds(j * CHUNK, CHUNK)]], gbuf.at[b],
                    gsem[b]).wait()
                convert(b)
                fire_scatter(b, j)
                @pl.when(g < ng - 1)
                def _(b=b, j=j):
                    fire_gather(b, j + ABUF)
            return carry
        lax.fori_loop(0, ng, group, 0)
        for b in range(ABUF):
            drain_scatter(b, (ng - 1) * ABUF + b)
        plsc.subcore_barrier()

        def piece(o, n):
            pltpu.sync_copy(acc.at[pl.ds(o, n)], fbuf.at[0, pl.ds(0, n)])
            pltpu.sync_copy(fbuf.at[0, pl.ds(0, n)], out.at[c, pl.ds(o, n)])
        _stripe_pieces(piece, s)

    return agg_kernel


_deg_kernel = _make_deg_kernel()
_agg64 = _make_agg_kernel(H1 // NC)
_agg32 = _make_agg_kernel(H2 // NC)

BR = 1000
GRID = N // BR


def _dinv_block(degp_ref):
    p = degp_ref[...]
    return lax.rsqrt(1.0 + p[0, 0] + p[0, 1])


def _pack_half(hp_half):
    w = hp_half.shape[1] // 2
    lo = lax.bitcast_convert_type(
        hp_half[:, :w].astype(jnp.bfloat16), jnp.uint16).astype(jnp.int32)
    hi = lax.bitcast_convert_type(
        hp_half[:, w:].astype(jnp.bfloat16), jnp.uint16).astype(jnp.int32)
    return lo | (hi << 16)


def _split_pack(hp, of_ref, op_ref):
    d = hp.shape[1] // 2
    h0, h1 = hp[:, :d], hp[:, d:]
    of_ref[0] = h0
    of_ref[1] = h1
    op_ref[0] = _pack_half(h0)
    op_ref[1] = _pack_half(h1)


def _layer1_body(x_ref, w_ref, degp_ref, of_ref, op_ref):
    dinv = _dinv_block(degp_ref)
    h = jnp.dot(x_ref[...], w_ref[...], preferred_element_type=jnp.float32)
    _split_pack(h * dinv[:, None], of_ref, op_ref)


def _layer2_body(q_ref, hp_ref, degp_ref, w_ref, of_ref, op_ref):
    dinv = _dinv_block(degp_ref)
    agg = jnp.concatenate([q_ref[0] + hp_ref[0], q_ref[1] + hp_ref[1]],
                          axis=1)
    z1 = jnp.maximum(agg * dinv[:, None], 0.0)
    h2 = jnp.dot(z1, w_ref[...], preferred_element_type=jnp.float32)
    _split_pack(h2 * dinv[:, None], of_ref, op_ref)


def _final_body(r_ref, hp_ref, degp_ref, o_ref):
    dinv = _dinv_block(degp_ref)
    agg = jnp.concatenate([r_ref[0] + hp_ref[0], r_ref[1] + hp_ref[1]],
                          axis=1)
    o_ref[...] = agg * dinv[:, None]


def _rows_spec(D):
    return pl.BlockSpec((BR, D), lambda i: (i, 0))


def _split_spec(DH):
    return pl.BlockSpec((NC, BR, DH), lambda i: (0, i, 0))


_deg_spec = pl.BlockSpec((1, NC, BR), lambda i: (i, 0, 0))


def _full_spec(shape):
    return pl.BlockSpec(shape, lambda i: tuple(0 for _ in shape))


def kernel(x, edge_index, W1, W2):
    npad = E_PAD - E
    src2d = jnp.concatenate(
        [edge_index[0], jnp.zeros((npad,), edge_index.dtype)]
    ).reshape(NS, CPS * CHUNK)
    dst2d = jnp.concatenate(
        [edge_index[1], jnp.full((npad,), N, edge_index.dtype)]
    ).reshape(NS, CPS * CHUNK)
    dst3d_w = dst2d.reshape(NW, CPS // NC, CHUNK)

    degp0, degp1 = _deg_kernel(dst3d_w)
    degp = jnp.stack([degp0.reshape(GRID, BR), degp1.reshape(GRID, BR)],
                     axis=1)

    h1p, h1pk = pl.pallas_call(
        _layer1_body,
        grid=(GRID,),
        in_specs=[_rows_spec(D_IN), _full_spec((D_IN, H1)), _deg_spec],
        out_specs=(_split_spec(H1 // NC), _split_spec(H1 // (2 * NC))),
        out_shape=(jax.ShapeDtypeStruct((NC, N, H1 // NC), jnp.float32),
                   jax.ShapeDtypeStruct((NC, N, H1 // (2 * NC)), jnp.int32)),
    )(x, W1, degp)

    q = _agg64(h1pk, src2d, dst2d)

    h2p, h2pk = pl.pallas_call(
        _layer2_body,
        grid=(GRID,),
        in_specs=[_split_spec(H1 // NC), _split_spec(H1 // NC),
                  _deg_spec, _full_spec((H1, H2))],
        out_specs=(_split_spec(H2 // NC), _split_spec(H2 // (2 * NC))),
        out_shape=(jax.ShapeDtypeStruct((NC, N, H2 // NC), jnp.float32),
                   jax.ShapeDtypeStruct((NC, N, H2 // (2 * NC)), jnp.int32)),
    )(q, h1p, degp, W2)

    r = _agg32(h2pk, src2d, dst2d)

    z = pl.pallas_call(
        _final_body,
        grid=(GRID,),
        in_specs=[_split_spec(H2 // NC), _split_spec(H2 // NC),
                  _deg_spec],
        out_specs=_rows_spec(H2),
        out_shape=jax.ShapeDtypeStruct((N, H2), jnp.float32),
    )(r, h2p, degp)

    return z
